# baseline (speedup 1.0000x reference)
import functools

import jax
import jax.numpy as jnp
from jax.experimental import pallas as pl
from jax.experimental.pallas import tpu as pltpu

TOPK = 10
KT = 512
WC = 128
FPAD = 2e6
FBIG = 4e6


def _topk_body(qc_ref, qs_ref, qcd_ref, qsd_ref, cf_ref, sf_ref, cd_ref,
               sd_ref, ov_ref, oi_ref, qcn, qsn, rv, ri,
               *, n_tiles, k_total, q, d):
    t = pl.program_id(0)

    @pl.when(t == 0)
    def _init():
        qcn[...] = qc_ref[...] / qcd_ref[...]
        qsn[...] = qs_ref[...] / qsd_ref[...]
        rv[...] = jnp.full((q, WC), -jnp.inf, jnp.float32)
        ri[...] = jnp.full((q, WC), FPAD, jnp.float32)

    cfn = cf_ref[...] / cd_ref[...]
    sfn = sf_ref[...] / sd_ref[...]

    dn = (((1,), (1,)), ((), ()))
    s = (100.0 * jax.lax.dot_general(qcn[...], cfn, dn)
         + jax.lax.dot_general(qsn[...], sfn, dn))

    col0 = t * KT
    fcol0 = col0.astype(jnp.float32)
    flane = jax.lax.broadcasted_iota(jnp.int32, (q, KT), 1).astype(jnp.float32)
    s = jnp.where(flane < (k_total - col0).astype(jnp.float32), s, -jnp.inf)

    work = jnp.concatenate([s, rv[...]], axis=1)
    flidx = jnp.concatenate([flane, ri[...] - fcol0], axis=1)

    nvs, nis = [], []
    for j in range(TOPK):
        m = jnp.max(work, axis=1, keepdims=True)
        p = jnp.min(jnp.where(work == m, flidx, FBIG), axis=1, keepdims=True)
        nvs.append(m)
        nis.append(p)
        if j + 1 < TOPK:
            work = jnp.where(flidx == p, -jnp.inf, work)
    nv = jnp.concatenate(nvs, axis=1)
    ni = jnp.concatenate(nis, axis=1) + fcol0
    rv[:, 0:TOPK] = nv
    ri[:, 0:TOPK] = ni

    @pl.when(t == n_tiles - 1)
    def _flush():
        ov_ref[...] = nv
        oi_ref[...] = ni.astype(jnp.int32)


def kernel(query_clip, query_sbert, clip_features, sbert_features, k):
    q, d = query_clip.shape
    k_total = clip_features.shape[0]
    n_tiles = pl.cdiv(k_total, KT)

    qcd = jnp.linalg.norm(query_clip, axis=-1, keepdims=True) + 1e-8
    qsd = jnp.linalg.norm(query_sbert, axis=-1, keepdims=True) + 1e-8
    cd = jnp.linalg.norm(clip_features, axis=-1, keepdims=True) + 1e-8
    sd = jnp.linalg.norm(sbert_features, axis=-1, keepdims=True) + 1e-8

    body = functools.partial(_topk_body, n_tiles=n_tiles, k_total=k_total,
                             q=q, d=d)
    vals, idx = pl.pallas_call(
        body,
        grid=(n_tiles,),
        in_specs=[
            pl.BlockSpec((q, d), lambda t: (0, 0)),
            pl.BlockSpec((q, d), lambda t: (0, 0)),
            pl.BlockSpec((q, 1), lambda t: (0, 0)),
            pl.BlockSpec((q, 1), lambda t: (0, 0)),
            pl.BlockSpec((KT, d), lambda t: (t, 0)),
            pl.BlockSpec((KT, d), lambda t: (t, 0)),
            pl.BlockSpec((KT, 1), lambda t: (t, 0)),
            pl.BlockSpec((KT, 1), lambda t: (t, 0)),
        ],
        out_specs=[
            pl.BlockSpec((q, TOPK), lambda t: (0, 0)),
            pl.BlockSpec((q, TOPK), lambda t: (0, 0)),
        ],
        out_shape=[
            jax.ShapeDtypeStruct((q, TOPK), jnp.float32),
            jax.ShapeDtypeStruct((q, TOPK), jnp.int32),
        ],
        scratch_shapes=[
            pltpu.VMEM((q, d), jnp.float32),
            pltpu.VMEM((q, d), jnp.float32),
            pltpu.VMEM((q, WC), jnp.float32),
            pltpu.VMEM((q, WC), jnp.float32),
        ],
        compiler_params=pltpu.CompilerParams(
            dimension_semantics=("arbitrary",),
        ),
    )(query_clip, query_sbert, qcd, qsd, clip_features, sbert_features, cd, sd)
    return vals, idx


# speedup vs baseline: 1.2105x; 1.2105x over previous
import functools

import jax
import jax.numpy as jnp
from jax.experimental import pallas as pl
from jax.experimental.pallas import tpu as pltpu

TOPK = 10
KT = 1024
WC = 128
FPAD = 2e6
FBIG = 4e6


def _topk_body(qc_ref, qs_ref, qcd_ref, qsd_ref, cf_ref, sf_ref, cd_ref,
               sd_ref, ov_ref, oi_ref, qcn, qsn, rv, ri, sprev,
               *, n_tiles, k_total, q, d):
    t = pl.program_id(0)

    @pl.when(t == 0)
    def _init():
        qcn[...] = qc_ref[...] / qcd_ref[...]
        qsn[...] = qs_ref[...] / qsd_ref[...]

    first = t == 0
    sold = jnp.where(first, -jnp.inf, sprev[...])
    rvo = jnp.where(first, -jnp.inf, rv[...])
    rio = jnp.where(first, FPAD, ri[...])

    flane = jax.lax.broadcasted_iota(jnp.int32, (q, KT), 1).astype(jnp.float32)

    cfn = cf_ref[...] / cd_ref[...]
    sfn = sf_ref[...] / sd_ref[...]
    dn = (((1,), (1,)), ((), ()))
    s = (100.0 * jax.lax.dot_general(qcn[...], cfn, dn)
         + jax.lax.dot_general(qsn[...], sfn, dn))
    s = jnp.where(flane < (k_total - t * KT).astype(jnp.float32), s, -jnp.inf)

    col0 = (t - 1) * KT
    fcol0 = col0.astype(jnp.float32)

    work = jnp.concatenate([sold, rvo], axis=1)
    flidx = jnp.concatenate([flane, rio - fcol0], axis=1)

    nvs, nis = [], []
    for j in range(TOPK):
        m = jnp.max(work, axis=1, keepdims=True)
        p = jnp.min(jnp.where(work == m, flidx, FBIG), axis=1, keepdims=True)
        nvs.append(m)
        nis.append(p)
        if j + 1 < TOPK:
            work = jnp.where(flidx == p, -jnp.inf, work)
    nv = jnp.concatenate(nvs, axis=1)
    ni = jnp.concatenate(nis, axis=1) + fcol0
    rv[:, 0:TOPK] = nv
    ri[:, 0:TOPK] = ni
    sprev[...] = s

    @pl.when(t == 0)
    def _pad_tail():
        rv[:, TOPK:] = jnp.full((q, WC - TOPK), -jnp.inf, jnp.float32)
        ri[:, TOPK:] = jnp.full((q, WC - TOPK), FPAD, jnp.float32)

    @pl.when(t == n_tiles)
    def _flush():
        ov_ref[...] = nv
        oi_ref[...] = ni.astype(jnp.int32)


def kernel(query_clip, query_sbert, clip_features, sbert_features, k):
    q, d = query_clip.shape
    k_total = clip_features.shape[0]
    n_tiles = pl.cdiv(k_total, KT)

    qcd = jnp.linalg.norm(query_clip, axis=-1, keepdims=True) + 1e-8
    qsd = jnp.linalg.norm(query_sbert, axis=-1, keepdims=True) + 1e-8
    cd = jnp.linalg.norm(clip_features, axis=-1, keepdims=True) + 1e-8
    sd = jnp.linalg.norm(sbert_features, axis=-1, keepdims=True) + 1e-8

    body = functools.partial(_topk_body, n_tiles=n_tiles, k_total=k_total,
                             q=q, d=d)
    kt_idx = lambda t: (jnp.minimum(t, n_tiles - 1), 0)
    vals, idx = pl.pallas_call(
        body,
        grid=(n_tiles + 1,),
        in_specs=[
            pl.BlockSpec((q, d), lambda t: (0, 0)),
            pl.BlockSpec((q, d), lambda t: (0, 0)),
            pl.BlockSpec((q, 1), lambda t: (0, 0)),
            pl.BlockSpec((q, 1), lambda t: (0, 0)),
            pl.BlockSpec((KT, d), kt_idx),
            pl.BlockSpec((KT, d), kt_idx),
            pl.BlockSpec((KT, 1), kt_idx),
            pl.BlockSpec((KT, 1), kt_idx),
        ],
        out_specs=[
            pl.BlockSpec((q, TOPK), lambda t: (0, 0)),
            pl.BlockSpec((q, TOPK), lambda t: (0, 0)),
        ],
        out_shape=[
            jax.ShapeDtypeStruct((q, TOPK), jnp.float32),
            jax.ShapeDtypeStruct((q, TOPK), jnp.int32),
        ],
        scratch_shapes=[
            pltpu.VMEM((q, d), jnp.float32),
            pltpu.VMEM((q, d), jnp.float32),
            pltpu.VMEM((q, WC), jnp.float32),
            pltpu.VMEM((q, WC), jnp.float32),
            pltpu.VMEM((q, KT), jnp.float32),
        ],
        compiler_params=pltpu.CompilerParams(
            dimension_semantics=("arbitrary",),
        ),
    )(query_clip, query_sbert, qcd, qsd, clip_features, sbert_features, cd, sd)
    return vals, idx
